# baseline (speedup 1.0000x reference)
import jax
import jax.numpy as jnp
from jax import lax
from jax.experimental import pallas as pl
from jax.experimental.pallas import tpu as pltpu
from jax.experimental.pallas import tpu_sc as plsc

N_NODES = 10000
N_EDGES = 320000
D = 128
DH = D // 2

NC = 2
NS = 16

EDGES_PER_TILE = N_EDGES // NS
CHUNK = 125
CHUNKS_PER_TILE = EDGES_PER_TILE // CHUNK
NBUF = 4
N_PAD = 10240
ROWS_PER_TILE = N_PAD // NS
COPY_ROWS = 128
COPY_STEPS = ROWS_PER_TILE // COPY_ROWS


def _sc_agg_body(feat_hbm, src_hbm, dst_hbm, out_hbm,
                 src_v, dst_v, rows, stage_v, agg_sh, gsems, ssems):
    c = lax.axis_index("c")
    s = lax.axis_index("s")

    zeros16 = jnp.zeros((16,), jnp.float32)

    def _zero_row(i, carry):
        for j in range(DH // 16):
            stage_v[i, pl.ds(j * 16, 16)] = zeros16
        return carry

    lax.fori_loop(0, COPY_ROWS, _zero_row, 0)
    for p in range(COPY_STEPS):
        pltpu.sync_copy(stage_v, agg_sh.at[pl.ds(s * ROWS_PER_TILE + p * COPY_ROWS, COPY_ROWS)])

    pltpu.sync_copy(src_hbm.at[c, s], src_v)
    pltpu.sync_copy(dst_hbm.at[s], dst_v)

    plsc.subcore_barrier()

    def _gstart(i, b):
        pltpu.async_copy(feat_hbm.at[src_v.at[i]], rows[b], gsems[b])

    def _gwait(b):
        pltpu.make_async_copy(feat_hbm.at[src_v.at[0]], rows[b], gsems[b]).wait()

    def _sstart(i, b):
        pltpu.async_copy(rows[b], agg_sh.at[dst_v.at[i]], ssems[b], add=True)

    def _swait(b):
        pltpu.make_async_copy(rows[b], agg_sh.at[dst_v.at[0]], ssems[b]).wait()

    last = CHUNKS_PER_TILE - 1
    for b in range(NBUF):
        _gstart(b, b)

    def _group(g, carry):
        base = NBUF * g
        for b in range(NBUF):
            _gwait(b)
            _sstart(base + b, b)
        for b in range(NBUF):
            _swait(b)
            _gstart(jnp.minimum(base + NBUF + b, last), b)
        return carry

    lax.fori_loop(0, CHUNKS_PER_TILE // NBUF, _group, 0)
    for b in range(NBUF):
        _gwait(b)

    plsc.subcore_barrier()

    for p in range(COPY_STEPS):
        base = s * ROWS_PER_TILE + p * COPY_ROWS
        pltpu.sync_copy(agg_sh.at[pl.ds(base, COPY_ROWS)], stage_v)
        pltpu.sync_copy(stage_v, out_hbm.at[c, pl.ds(base, COPY_ROWS)])


@jax.jit
def _sc_aggregate(feat2, src_r, dst_r):
    mesh = plsc.VectorSubcoreMesh(core_axis_name="c", subcore_axis_name="s")
    return pl.kernel(
        _sc_agg_body,
        out_type=jax.ShapeDtypeStruct((NC, N_PAD, DH), jnp.float32),
        mesh=mesh,
        scratch_types=[
            pltpu.VMEM((CHUNKS_PER_TILE, CHUNK), jnp.int32),
            pltpu.VMEM((CHUNKS_PER_TILE, CHUNK), jnp.int32),
            [pltpu.VMEM((CHUNK, DH), jnp.float32)] * NBUF,
            pltpu.VMEM((COPY_ROWS, DH), jnp.float32),
            pltpu.VMEM_SHARED((N_PAD, DH), jnp.float32),
            [pltpu.SemaphoreType.DMA] * NBUF,
            [pltpu.SemaphoreType.DMA] * NBUF,
        ],
        compiler_params=pltpu.CompilerParams(use_tc_tiling_on_sc=False),
    )(feat2, src_r, dst_r)


def _tc_mm1_body(feat_ref, w1_ref, out_ref):
    out_ref[...] = jnp.dot(feat_ref[...], w1_ref[...],
                           preferred_element_type=jnp.float32)


@jax.jit
def _tc_mm1(feat, W1):
    blk = 1000
    return pl.pallas_call(
        _tc_mm1_body,
        grid=(N_NODES // blk,),
        in_specs=[
            pl.BlockSpec((blk, D), lambda i: (i, 0)),
            pl.BlockSpec((D, D), lambda i: (0, 0)),
        ],
        out_specs=pl.BlockSpec((blk, D), lambda i: (i, 0)),
        out_shape=jax.ShapeDtypeStruct((N_NODES, D), jnp.float32),
    )(feat, W1)


def _tc_mm2_body(h1_ref, p_ref, w2_ref, out_ref):
    agg = jnp.concatenate([p_ref[0], p_ref[1]], axis=-1)
    out_ref[...] = h1_ref[...] + jnp.dot(agg, w2_ref[...],
                                         preferred_element_type=jnp.float32)


@jax.jit
def _tc_mm2(h1, partials, W2):
    blk = 1000
    return pl.pallas_call(
        _tc_mm2_body,
        grid=(N_NODES // blk,),
        in_specs=[
            pl.BlockSpec((blk, D), lambda i: (i, 0)),
            pl.BlockSpec((NC, blk, DH), lambda i: (0, i, 0)),
            pl.BlockSpec((D, D), lambda i: (0, 0)),
        ],
        out_specs=pl.BlockSpec((blk, D), lambda i: (i, 0)),
        out_shape=jax.ShapeDtypeStruct((N_NODES, D), jnp.float32),
    )(h1, partials, W2)


def kernel(feat, edge_index, W1, W2):
    edge_index = edge_index.astype(jnp.int32)
    src2 = edge_index[0] * 2
    src_r = (src2[None, :] + jnp.arange(NC, dtype=jnp.int32)[:, None]).reshape(
        NC, NS, CHUNKS_PER_TILE, CHUNK)
    dst_r = edge_index[1].reshape(NS, CHUNKS_PER_TILE, CHUNK)
    feat2 = feat.reshape(N_NODES * 2, DH)
    partials = _sc_aggregate(feat2, src_r, dst_r)
    h1 = _tc_mm1(feat, W1)
    return _tc_mm2(h1, partials, W2)
